# baseline (speedup 1.0000x reference)
import functools

import jax
import jax.numpy as jnp
from jax import lax
from jax.experimental import pallas as pl
from jax.experimental.pallas import tpu as pltpu
from jax.experimental.pallas import tpu_sc as plsc

_B = 4096
_L = 200
_D = 64
_EPS = 1e-12

_NR = 2 * _B
_CH = 112
_NCH = 2
_LP = _CH * _NCH
_NW = 32
_RW = _NR // _NW
_LANES = 16
_KD = _D // _LANES
_UNROLL = 8


def _sc_pool_body(idx_hbm, table_hbm, out_hbm, idxv, buf, outbuf, t0v,
                  sem0, sem1):
    wid = lax.axis_index("s") * 2 + lax.axis_index("c")
    base = wid * _RW
    sems = (sem0, sem1)

    pltpu.sync_copy(table_hbm.at[0], t0v)
    t0 = [t0v[pl.ds(k * _LANES, _LANES)] for k in range(_KD)]

    def gather_descs(slot):
        return [
            pltpu.make_async_copy(
                table_hbm.at[idxv.at[slot, j]],
                buf.at[slot, pl.ds(j * _CH, _CH)],
                sems[slot],
            )
            for j in range(_NCH)
        ]

    def fire(slot, row):
        pltpu.sync_copy(idx_hbm.at[row], idxv.at[slot])
        for dsc in gather_descs(slot):
            dsc.start()

    def consume(slot, local_row):
        for dsc in gather_descs(slot):
            dsc.wait()
        one = jnp.ones((_LANES,), jnp.float32)
        zv = jnp.zeros((_LANES,), jnp.float32)
        nzv = jnp.zeros((_LANES,), jnp.float32)
        for j in range(_NCH):
            for c in range(_CH // _LANES):
                v = idxv[slot, j, pl.ds(c * _LANES, _LANES)]
                nzv = nzv + jnp.where(v == 0, one, zv)
        nzf = jnp.broadcast_to(jnp.sum(nzv), (_LANES,))
        cnt = jnp.float32(_LP) - nzf

        def acc_body(g, accs):
            accs = list(accs)
            for u in range(_UNROLL):
                r = g * _UNROLL + u
                for k in range(_KD):
                    accs[k] = accs[k] + buf[slot, r,
                                            pl.ds(k * _LANES, _LANES)]
            return tuple(accs)

        zero = jnp.zeros((_LANES,), jnp.float32)
        accs = lax.fori_loop(0, _LP // _UNROLL, acc_body, (zero,) * _KD)
        inv = 1.0 / cnt
        for k in range(_KD):
            outbuf[local_row, pl.ds(k * _LANES, _LANES)] = (
                (accs[k] - nzf * t0[k]) * inv)

    fire(0, base)

    def outer(i, carry):
        for phase in range(2):
            local = 2 * i + phase
            nxt = local + 1

            @pl.when(nxt < _RW)
            def _():
                fire(1 - phase, base + nxt)

            consume(phase, local)
        return carry

    lax.fori_loop(0, _RW // 2, outer, 0)
    pltpu.sync_copy(outbuf, out_hbm.at[pl.ds(base, _RW)])


_sc_pool = functools.partial(
    pl.kernel,
    mesh=plsc.VectorSubcoreMesh(core_axis_name="c", subcore_axis_name="s"),
    compiler_params=pltpu.CompilerParams(
        needs_layout_passes=False, use_tc_tiling_on_sc=False),
    out_type=jax.ShapeDtypeStruct((_NR, _D), jnp.float32),
    scratch_types=[
        pltpu.VMEM((2, _NCH, _CH), jnp.int32),
        pltpu.VMEM((2, _LP, _D), jnp.float32),
        pltpu.VMEM((_RW, _D), jnp.float32),
        pltpu.VMEM((_D,), jnp.float32),
        pltpu.SemaphoreType.DMA,
        pltpu.SemaphoreType.DMA,
    ],
)(_sc_pool_body)


def _ln_body(x_ref, g_ref, b_ref, o_ref):
    x = x_ref[...]
    mu = jnp.mean(x, axis=-1, keepdims=True)
    xc = x - mu
    var = jnp.mean(xc * xc, axis=-1, keepdims=True)
    o_ref[...] = xc * lax.rsqrt(var + _EPS) * g_ref[...] + b_ref[...]


def _layernorm(pooled, gamma, beta):
    blk = 1024
    return pl.pallas_call(
        _ln_body,
        grid=(_NR // blk,),
        in_specs=[
            pl.BlockSpec((blk, _D), lambda i: (i, 0)),
            pl.BlockSpec((1, _D), lambda i: (0, 0)),
            pl.BlockSpec((1, _D), lambda i: (0, 0)),
        ],
        out_specs=pl.BlockSpec((blk, _D), lambda i: (i, 0)),
        out_shape=jax.ShapeDtypeStruct((_NR, _D), jnp.float32),
    )(pooled, gamma, beta)


def kernel(x_s, x_t, table, gamma, beta):
    idx = jnp.concatenate(
        [x_s.astype(jnp.int32), x_t.astype(jnp.int32)], axis=0)
    idx = jnp.pad(idx, ((0, 0), (0, _LP - _L)))
    idx = idx.reshape(_NR, _NCH, _CH)
    pooled = _sc_pool(idx, table)
    out = _layernorm(pooled, gamma.reshape(1, _D), beta.reshape(1, _D))
    return out[:_B], out[_B:]
